# Initial kernel scaffold; baseline (speedup 1.0000x reference)
import jax
import jax.numpy as jnp
from jax.experimental import pallas as pl


def kernel(x, edge_index, batch, params):
    raise NotImplementedError("write your pallas kernel here")


# speedup vs baseline: 1.0046x; 1.0046x over previous
import functools

import jax
import jax.numpy as jnp
from jax.experimental import pallas as pl
from jax.experimental.pallas import tpu as pltpu

_DIMS = [(128, 512), (512, 256), (256, 64), (64, 32)]
_G = 16
_OUT = 9


def _pool_fc_kernel(h_ref, batch_ref, gw_ref, gb_ref, fw_ref, fb_ref, o_ref):
    h = h_ref[...]
    batch = batch_ref[...]
    gate = jax.lax.dot_general(
        gw_ref[...], h, dimension_numbers=(((1,), (1,)), ((), ())),
        preferred_element_type=jnp.float32,
    ) + gb_ref[...]
    onehot = (batch == jax.lax.broadcasted_iota(jnp.int32, (_G, 1), 0)).astype(
        jnp.float32
    )
    neg = jnp.float32(-1e30)
    gmax = jnp.max(jnp.where(onehot > 0, gate, neg), axis=1, keepdims=True)
    gmax = jnp.where(gmax > neg * 0.5, gmax, 0.0)
    ge = jnp.exp(gate - gmax) * onehot
    gden = jnp.sum(ge, axis=1, keepdims=True)
    w = ge / (gden + 1e-16)
    pooled = jnp.dot(w, h, preferred_element_type=jnp.float32)
    o_ref[...] = jnp.dot(pooled, fw_ref[...].T,
                         preferred_element_type=jnp.float32) + fb_ref[...]


def _pool_fc(h, batch, gw, gb, fw, fb):
    return pl.pallas_call(
        _pool_fc_kernel,
        out_shape=jax.ShapeDtypeStruct((_G, _OUT), jnp.float32),
    )(h, batch[None, :], gw, gb[None, :], fw, fb[None, :])


def _conv(h, src, dst, params, i, dout):
    n = h.shape[0]
    q = h @ params['c%d_Wq' % i].T + params['c%d_bq' % i]
    k = h @ params['c%d_Wk' % i].T + params['c%d_bk' % i]
    v = h @ params['c%d_Wv' % i].T + params['c%d_bv' % i]
    alpha = jnp.sum(q[dst] * k[src], axis=-1) / jnp.sqrt(float(dout))
    amax = jax.ops.segment_max(alpha, dst, num_segments=n)
    amax = jnp.where(jnp.isfinite(amax), amax, 0.0)
    ex = jnp.exp(alpha - amax[dst])
    denom = jax.ops.segment_sum(ex, dst, num_segments=n)
    attn = ex / (denom[dst] + 1e-16)
    out = jax.ops.segment_sum(v[src] * attn[:, None], dst, num_segments=n)
    out = out + h @ params['c%d_Ws' % i].T + params['c%d_bs' % i]
    return out


def kernel(x, edge_index, batch, params):
    src = edge_index[0]
    dst = edge_index[1]
    h = x
    for i, (din, dout) in enumerate(_DIMS):
        h = jax.nn.elu(_conv(h, src, dst, params, i, dout))
    return _pool_fc(h, batch, params['gate_W'], params['gate_b'],
                    params['fc_W'], params['fc_b'])


# speedup vs baseline: 1.2402x; 1.2345x over previous
import functools

import jax
import jax.numpy as jnp
from jax import lax
from jax.experimental import pallas as pl
from jax.experimental.pallas import tpu as pltpu
from jax.experimental.pallas import tpu_sc as plsc

_DIMS = [(128, 512), (512, 256), (256, 64), (64, 32)]
_G = 16
_OUT = 9
_N = 10000
_E = 320000
_CN = 64
_NCH = (_N + _CN - 1) // _CN
_NPAD = _NCH * _CN
_NCHP = 160
_B = 64
_NW = 32
_ROWS = _CN + 16
_BN = _NPAD // 8


def _proj_body(h_ref, wq, wk, wv, bq, bk, bv, q_ref, k_ref, v_ref):
    h = h_ref[...]
    dn = (((1,), (1,)), ((), ()))
    q_ref[...] = lax.dot_general(h, wq[...], dn,
                                 preferred_element_type=jnp.float32) + bq[...]
    k_ref[...] = lax.dot_general(h, wk[...], dn,
                                 preferred_element_type=jnp.float32) + bk[...]
    v_ref[...] = lax.dot_general(h, wv[...], dn,
                                 preferred_element_type=jnp.float32) + bv[...]


@functools.partial(jax.jit, static_argnames=("din", "dout"))
def _proj(h, wq, wk, wv, bq, bk, bv, din, dout):
    blk = lambda i: (i, 0)
    full = lambda i: (0, 0)
    return pl.pallas_call(
        _proj_body,
        grid=(8,),
        in_specs=[
            pl.BlockSpec((_BN, din), blk),
            pl.BlockSpec((dout, din), full),
            pl.BlockSpec((dout, din), full),
            pl.BlockSpec((dout, din), full),
            pl.BlockSpec((1, dout), full),
            pl.BlockSpec((1, dout), full),
            pl.BlockSpec((1, dout), full),
        ],
        out_specs=[
            pl.BlockSpec((_BN, dout), blk),
            pl.BlockSpec((_BN, dout), blk),
            pl.BlockSpec((_BN, dout), blk),
        ],
        out_shape=[jax.ShapeDtypeStruct((_NPAD, dout), jnp.float32)] * 3,
    )(h, wq, wk, wv, bq[None, :], bk[None, :], bv[None, :])


def _post_body(agg_ref, h_ref, ws, bs, o_ref):
    h = h_ref[...]
    dn = (((1,), (1,)), ((), ()))
    s = lax.dot_general(h, ws[...], dn,
                        preferred_element_type=jnp.float32) + bs[...]
    x = agg_ref[...] + s
    o_ref[...] = jnp.where(x > 0, x, jnp.exp(x) - 1.0)


@functools.partial(jax.jit, static_argnames=("din", "dout"))
def _post(agg, h, ws, bs, din, dout):
    blk = lambda i: (i, 0)
    full = lambda i: (0, 0)
    return pl.pallas_call(
        _post_body,
        grid=(8,),
        in_specs=[
            pl.BlockSpec((_BN, dout), blk),
            pl.BlockSpec((_BN, din), blk),
            pl.BlockSpec((dout, din), full),
            pl.BlockSpec((1, dout), full),
        ],
        out_specs=pl.BlockSpec((_BN, dout), blk),
        out_shape=jax.ShapeDtypeStruct((_NPAD, dout), jnp.float32),
    )(agg, h, ws, bs[None, :])


def _pool_fc_body(h_ref, batch_ref, gw_ref, gb_ref, fw_ref, fb_ref, o_ref):
    h = h_ref[...]
    batch = batch_ref[...]
    gate = lax.dot_general(
        gw_ref[...], h, dimension_numbers=(((1,), (1,)), ((), ())),
        preferred_element_type=jnp.float32,
    ) + gb_ref[...]
    onehot = (batch == lax.broadcasted_iota(jnp.int32, (_G, 1), 0)).astype(
        jnp.float32)
    neg = jnp.float32(-1e30)
    gmax = jnp.max(jnp.where(onehot > 0, gate, neg), axis=1, keepdims=True)
    gmax = jnp.where(gmax > neg * 0.5, gmax, 0.0)
    ge = jnp.exp(gate - gmax) * onehot
    gden = jnp.sum(ge, axis=1, keepdims=True)
    w = ge / (gden + 1e-16)
    pooled = jnp.dot(w, h, preferred_element_type=jnp.float32)
    o_ref[...] = jnp.dot(pooled, fw_ref[...].T,
                         preferred_element_type=jnp.float32) + fb_ref[...]


def _pool_fc(h, batch, gw, gb, fw, fb):
    return pl.pallas_call(
        _pool_fc_body,
        out_shape=jax.ShapeDtypeStruct((_G, _OUT), jnp.float32),
    )(h, batch[None, :], gw, gb[None, :], fw, fb[None, :])


def _tshift(v, idx):
    return jnp.take_along_axis(v, idx, axis=0, mode="promise_in_bounds")


def _make_edge_kernel(dout):
    scale = 1.0 / float(dout) ** 0.5
    nf = dout // 16
    mesh = plsc.VectorSubcoreMesh(core_axis_name="c", subcore_axis_name="s",
                                  num_cores=2, num_subcores=16)
    ep = _E + _B
    rounds = (_NCH + _NW - 1) // _NW

    @functools.partial(
        pl.kernel,
        out_type=[
            jax.ShapeDtypeStruct((_NPAD, dout), jnp.float32),
            jax.ShapeDtypeStruct((ep,), jnp.float32),
        ],
        mesh=mesh,
        compiler_params=pltpu.CompilerParams(
            needs_layout_passes=False, use_tc_tiling_on_sc=False),
        scratch_types=[
            pltpu.VMEM((_B, dout), jnp.float32),
            pltpu.VMEM((_B, dout), jnp.float32),
            pltpu.VMEM((_B,), jnp.int32),
            pltpu.VMEM((_B,), jnp.int32),
            pltpu.VMEM((_B,), jnp.int32),
            pltpu.VMEM((_B,), jnp.float32),
            pltpu.VMEM((_CN,), jnp.float32),
            pltpu.VMEM((_CN,), jnp.float32),
            pltpu.VMEM((_NCHP,), jnp.int32),
            pltpu.VMEM_SHARED((16 * _ROWS, dout), jnp.float32),
            pltpu.SemaphoreType.DMA,
            pltpu.SemaphoreType.DMA,
        ],
    )
    def edge_kernel(qh, kh, vh, ssrc, sdst, choff, out_hbm, alpha_hbm,
                    buf_a, buf_b, sidx, didx, dlbuf, abuf, amax, denom,
                    choffv, shacc, sem1, sem2):
        lanes = lax.iota(jnp.int32, 16)
        cid = lax.axis_index("c")
        sid = lax.axis_index("s")
        wid = sid * 2 + cid
        base_sp = sid * _ROWS

        pltpu.sync_copy(choff, choffv)

        def sload(ref, i):
            v = plsc.load_gather(ref, [jnp.full((16,), i, jnp.int32)])
            return jnp.max(v)

        for r in range(rounds):
            c = wid + _NW * r

            @pl.when(c < _NCH)
            def _chunk():
                eb = sload(choffv, c)
                ee = sload(choffv, c + 1)
                ab = pl.multiple_of((eb // _B) * _B, _B)
                nblk = (ee - ab + _B - 1) // _B

                @pl.loop(0, nblk)
                def _s1(j):
                    base = pl.multiple_of(ab + j * _B, _B)
                    pltpu.sync_copy(ssrc.at[pl.ds(base, _B)], sidx)
                    pltpu.sync_copy(sdst.at[pl.ds(base, _B)], didx)
                    cp1 = pltpu.async_copy(qh.at[didx], buf_a, sem1)
                    cp2 = pltpu.async_copy(kh.at[sidx], buf_b, sem2)
                    cp1.wait()
                    cp2.wait()
                    for t in range(_B // 16):
                        rows = lanes + 16 * t

                        def fbody(f, acc):
                            col = jnp.full((16,), f, jnp.int32)
                            qv = plsc.load_gather(buf_a, [rows, col])
                            kv = plsc.load_gather(buf_b, [rows, col])
                            return acc + qv * kv

                        acc = lax.fori_loop(0, dout, fbody,
                                            jnp.zeros((16,), jnp.float32),
                                            unroll=8)
                        abuf[pl.ds(16 * t, 16)] = acc * scale
                    pltpu.sync_copy(abuf, alpha_hbm.at[pl.ds(base, _B)])

                for u in range(_CN // 16):
                    amax[pl.ds(16 * u, 16)] = jnp.full((16,), -3.4e38,
                                                       jnp.float32)
                    denom[pl.ds(16 * u, 16)] = jnp.zeros((16,), jnp.float32)

                @pl.loop(0, nblk)
                def _s2a(j):
                    base = pl.multiple_of(ab + j * _B, _B)
                    pltpu.sync_copy(sdst.at[pl.ds(base, _B)], didx)
                    pltpu.sync_copy(alpha_hbm.at[pl.ds(base, _B)], abuf)
                    for t in range(_B // 16):
                        gid = base + 16 * t + lanes
                        valid = (gid >= eb) & (gid < ee)
                        d = didx[pl.ds(16 * t, 16)]
                        a = abuf[pl.ds(16 * t, 16)]
                        a = jnp.where(valid, a, jnp.float32(-3.4e38))
                        for s in (1, 2, 4, 8):
                            shl = jnp.maximum(lanes - s, 0)
                            pa = _tshift(a, shl)
                            pd = _tshift(d, shl)
                            ok = (lanes >= s) & (pd == d)
                            a = jnp.where(ok, jnp.maximum(a, pa), a)
                        nd = _tshift(d, jnp.minimum(lanes + 1, 15))
                        last = (lanes == 15) | (nd != d)
                        dl = jnp.clip(d - c * _CN, 0, _CN - 1)
                        cur = plsc.load_gather(amax, [dl])
                        plsc.store_scatter(amax, [dl], jnp.maximum(cur, a),
                                           mask=last & valid)

                @pl.loop(0, nblk)
                def _s2b(j):
                    base = pl.multiple_of(ab + j * _B, _B)
                    pltpu.sync_copy(sdst.at[pl.ds(base, _B)], didx)
                    pltpu.sync_copy(alpha_hbm.at[pl.ds(base, _B)], abuf)
                    for t in range(_B // 16):
                        gid = base + 16 * t + lanes
                        valid = (gid >= eb) & (gid < ee)
                        d = didx[pl.ds(16 * t, 16)]
                        a = abuf[pl.ds(16 * t, 16)]
                        dl = jnp.clip(d - c * _CN, 0, _CN - 1)
                        m = plsc.load_gather(amax, [dl])
                        ex = jnp.where(valid, jnp.exp(a - m), 0.0)
                        for s in (1, 2, 4, 8):
                            shl = jnp.maximum(lanes - s, 0)
                            pe = _tshift(ex, shl)
                            pd = _tshift(d, shl)
                            ok = (lanes >= s) & (pd == d)
                            ex = jnp.where(ok, ex + pe, ex)
                        nd = _tshift(d, jnp.minimum(lanes + 1, 15))
                        last = (lanes == 15) | (nd != d)
                        plsc.addupdate_scatter(denom, [dl], ex,
                                               mask=last & valid)

                @pl.loop(0, _B)
                def _z(rr):
                    for u in range(nf):
                        buf_b[rr, pl.ds(16 * u, 16)] = jnp.zeros(
                            (16,), jnp.float32)

                pltpu.sync_copy(buf_b, shacc.at[pl.ds(base_sp, _B)])
                pltpu.sync_copy(buf_b.at[pl.ds(0, 16)],
                                shacc.at[pl.ds(base_sp + _CN, 16)])

                @pl.loop(0, nblk)
                def _s3(j):
                    base = pl.multiple_of(ab + j * _B, _B)
                    pltpu.sync_copy(ssrc.at[pl.ds(base, _B)], sidx)
                    pltpu.sync_copy(sdst.at[pl.ds(base, _B)], didx)
                    pltpu.sync_copy(alpha_hbm.at[pl.ds(base, _B)], abuf)
                    cpv = pltpu.async_copy(vh.at[sidx], buf_a, sem1)
                    cpv.wait()
                    for t in range(_B // 16):
                        gid = base + 16 * t + lanes
                        valid = (gid >= eb) & (gid < ee)
                        d = didx[pl.ds(16 * t, 16)]
                        a = abuf[pl.ds(16 * t, 16)]
                        dlc = jnp.clip(d - c * _CN, 0, _CN - 1)
                        m = plsc.load_gather(amax, [dlc])
                        den = plsc.load_gather(denom, [dlc])
                        w = jnp.exp(a - m) / (den + 1e-16)
                        w = jnp.where(valid, w, 0.0)
                        dlbuf[pl.ds(16 * t, 16)] = base_sp + jnp.where(
                            valid, d - c * _CN, _CN)
                        rows = lanes + 16 * t

                        def fbody(f, _):
                            col = jnp.full((16,), f, jnp.int32)
                            vals = plsc.load_gather(buf_a, [rows, col])
                            plsc.store_scatter(buf_a, [rows, col], vals * w)
                            return 0

                        lax.fori_loop(0, dout, fbody, 0, unroll=8)
                    pltpu.sync_copy(buf_a, shacc.at[dlbuf], add=True)

                pltpu.sync_copy(shacc.at[pl.ds(base_sp, _CN)],
                                out_hbm.at[pl.ds(pl.multiple_of(c * _CN, _CN),
                                                 _CN)])

    return edge_kernel


_EDGE_KERNELS = {}


def _edge_kernel(dout):
    if dout not in _EDGE_KERNELS:
        _EDGE_KERNELS[dout] = _make_edge_kernel(dout)
    return _EDGE_KERNELS[dout]


def kernel(x, edge_index, batch, params):
    src = edge_index[0]
    dst = edge_index[1]
    perm = jnp.argsort(dst)
    sdst = dst[perm]
    ssrc = src[perm]
    ssrc_p = jnp.pad(ssrc, (0, _B))
    sdst_p = jnp.pad(sdst, (0, _B))
    bounds = jnp.arange(_NCHP, dtype=jnp.int32) * _CN
    choff = jnp.searchsorted(sdst, bounds).astype(jnp.int32)

    h = jnp.pad(x, ((0, _NPAD - _N), (0, 0)))
    batch_p = jnp.pad(batch, (0, _NPAD - _N), constant_values=_G)

    for i, (din, dout) in enumerate(_DIMS):
        q, k, v = _proj(h, params['c%d_Wq' % i], params['c%d_Wk' % i],
                        params['c%d_Wv' % i], params['c%d_bq' % i],
                        params['c%d_bk' % i], params['c%d_bv' % i],
                        din=din, dout=dout)
        agg, _ = _edge_kernel(dout)(q, k, v, ssrc_p, sdst_p, choff)
        h = _post(agg, h, params['c%d_Ws' % i], params['c%d_bs' % i],
                  din=din, dout=dout)

    return _pool_fc(h, batch_p, params['gate_W'], params['gate_b'],
                    params['fc_W'], params['fc_b'])
